# baseline (speedup 1.0000x reference)
import jax
import jax.numpy as jnp
from jax.experimental import pallas as pl
from jax.experimental.pallas import tpu as pltpu


def _make_pos_kernel(d, h, w):
    n = h * w

    def _pos_kernel(row_ref, col_ref, out_ref):
        lane = jax.lax.broadcasted_iota(jnp.int32, (w, n), 1)
        jrow = jax.lax.broadcasted_iota(jnp.int32, (w, n), 0)
        sel_top = (lane % w == jrow).astype(jnp.float32)
        sel_bot = (lane // w == jrow).astype(jnp.float32)
        out_ref[0, 0:d, :] = jax.lax.dot_general(
            col_ref[0:w, :], sel_top,
            dimension_numbers=(((0,), (0,)), ((), ())),
            preferred_element_type=jnp.float32)
        out_ref[0, d:2 * d, :] = jax.lax.dot_general(
            row_ref[0:h, :], sel_bot,
            dimension_numbers=(((0,), (0,)), ((), ())),
            preferred_element_type=jnp.float32)

    return _pos_kernel


def kernel(tensor, row_embed, col_embed):
    b = tensor.shape[0]
    h, w = tensor.shape[-2], tensor.shape[-1]
    d = row_embed.shape[1]

    out = pl.pallas_call(
        _make_pos_kernel(d, h, w),
        grid=(b,),
        in_specs=[
            pl.BlockSpec(row_embed.shape, lambda i: (0, 0)),
            pl.BlockSpec(col_embed.shape, lambda i: (0, 0)),
        ],
        out_specs=pl.BlockSpec((1, 2 * d, h * w), lambda i: (i, 0, 0)),
        out_shape=jax.ShapeDtypeStruct((b, 2 * d, h * w), jnp.float32),
        compiler_params=pltpu.CompilerParams(
            dimension_semantics=("parallel",)),
    )(row_embed, col_embed)
    return out.reshape(b, 2 * d, h, w)


# speedup vs baseline: 1.0689x; 1.0689x over previous
import jax
import jax.numpy as jnp
from jax.experimental import pallas as pl
from jax.experimental.pallas import tpu as pltpu


def _make_pos_kernel(b, d, h, w):
    n = h * w

    def _pos_kernel(row_ref, col_ref, out_ref, scratch_ref, sem):
        lane = jax.lax.broadcasted_iota(jnp.int32, (w, n), 1)
        jrow = jax.lax.broadcasted_iota(jnp.int32, (w, n), 0)
        sel_top = (lane % w == jrow).astype(jnp.float32)
        sel_bot = (lane // w == jrow).astype(jnp.float32)
        scratch_ref[0:d, :] = jax.lax.dot_general(
            col_ref[0:w, :], sel_top,
            dimension_numbers=(((0,), (0,)), ((), ())),
            preferred_element_type=jnp.float32)
        scratch_ref[d:2 * d, :] = jax.lax.dot_general(
            row_ref[0:h, :], sel_bot,
            dimension_numbers=(((0,), (0,)), ((), ())),
            preferred_element_type=jnp.float32)
        copies = [
            pltpu.make_async_copy(scratch_ref, out_ref.at[i], sem.at[i])
            for i in range(b)
        ]
        for c in copies:
            c.start()
        for c in copies:
            c.wait()

    return _pos_kernel


def kernel(tensor, row_embed, col_embed):
    b = tensor.shape[0]
    h, w = tensor.shape[-2], tensor.shape[-1]
    d = row_embed.shape[1]

    out = pl.pallas_call(
        _make_pos_kernel(b, d, h, w),
        in_specs=[
            pl.BlockSpec(row_embed.shape, lambda: (0, 0)),
            pl.BlockSpec(col_embed.shape, lambda: (0, 0)),
        ],
        out_specs=pl.BlockSpec(memory_space=pl.ANY),
        out_shape=jax.ShapeDtypeStruct((b, 2 * d, h * w), jnp.float32),
        scratch_shapes=[
            pltpu.VMEM((2 * d, h * w), jnp.float32),
            pltpu.SemaphoreType.DMA((b,)),
        ],
        compiler_params=pltpu.CompilerParams(skip_device_barrier=True),
    )(row_embed, col_embed)
    return out.reshape(b, 2 * d, h, w)
